# baseline (speedup 1.0000x reference)
import functools

import jax
import jax.numpy as jnp
from jax import lax
from jax.experimental import pallas as pl
from jax.experimental.pallas import tpu as pltpu
from jax.experimental.pallas import tpu_sc as plsc

B = 16384
S = 100
D1 = 512
D2 = 256
FID = 1000000

NSPLIT = 4
BCH = B // NSPLIT

NC = 2
NS = 16
NW = NC * NS
E = BCH * S
EW = E // NW
EH = EW // 2

IW = 25
CS = FID // IW
CH2 = CS // 4

_SC_PARAMS = pltpu.CompilerParams(use_tc_tiling_on_sc=False,
                                  needs_layout_passes=False)


@functools.cache
def _get_sc_kernels():
    mesh = plsc.VectorSubcoreMesh(core_axis_name="c", subcore_axis_name="s")

    @functools.partial(
        pl.kernel,
        mesh=mesh,
        out_type=jax.ShapeDtypeStruct((FID, 2), jnp.float32),
        scratch_types=[
            pltpu.VMEM((2 * CH2,), jnp.float32),
            pltpu.VMEM((CH2, 2), jnp.float32),
        ],
        compiler_params=_SC_PARAMS,
    )
    def _interleave(sparse_hbm, certain_hbm, t2, buf2, ibuf):
        wid = lax.axis_index("s") * NC + lax.axis_index("c")

        @pl.when(wid < IW)
        def _():
            lanes = lax.iota(jnp.int32, 16)
            zeros = lanes * 0
            ones = zeros + 1
            for hh in range(4):
                q0 = wid * CS + hh * CH2
                pltpu.sync_copy(sparse_hbm.at[pl.ds(q0, CH2)],
                                buf2.at[pl.ds(0, CH2)])
                pltpu.sync_copy(certain_hbm.at[pl.ds(q0, CH2)],
                                buf2.at[pl.ds(CH2, CH2)])

                def body(j, carry):
                    for u in range(5):
                        base = (j * 5 + u) * 16
                        rows = base + lanes
                        sv = buf2[pl.ds(base, 16)]
                        cv = buf2[pl.ds(CH2 + base, 16)]
                        plsc.store_scatter(ibuf, [rows, zeros], sv)
                        plsc.store_scatter(ibuf, [rows, ones], cv)
                    return carry

                lax.fori_loop(0, CH2 // 80, body, 0)
                pltpu.sync_copy(ibuf, t2.at[pl.ds(q0, CH2)])

    def _make_gather(base):
        @functools.partial(
            pl.kernel,
            mesh=mesh,
            out_type=(jax.ShapeDtypeStruct((E,), jnp.float32),
                      jax.ShapeDtypeStruct((E,), jnp.float32)),
            scratch_types=[
                pltpu.VMEM((EH,), jnp.int32),
                pltpu.VMEM((EH, 2), jnp.float32),
                pltpu.VMEM((EH,), jnp.float32),
                pltpu.VMEM((EH,), jnp.float32),
                pltpu.SemaphoreType.DMA,
            ],
            compiler_params=_SC_PARAMS,
        )
        def _gather(idx_hbm, t2_hbm, out_s, out_c,
                    idx_v, obuf, sflat, cflat, sem):
            wid = lax.axis_index("s") * NC + lax.axis_index("c")
            lanes = lax.iota(jnp.int32, 16)
            zeros = lanes * 0
            ones = zeros + 1
            for hh in range(2):
                r0 = wid * EW + hh * EH
                pltpu.sync_copy(idx_hbm.at[pl.ds(base + r0, EH)], idx_v)
                pltpu.async_copy(t2_hbm.at[idx_v], obuf, sem).wait()

                def body(j, carry):
                    for u in range(8):
                        bb = (j * 8 + u) * 16
                        rows = bb + lanes
                        sv = plsc.load_gather(obuf, [rows, zeros])
                        cv = plsc.load_gather(obuf, [rows, ones])
                        sflat[pl.ds(bb, 16)] = sv
                        cflat[pl.ds(bb, 16)] = cv
                    return carry

                lax.fori_loop(0, EH // 128, body, 0)
                pltpu.sync_copy(sflat, out_s.at[pl.ds(r0, EH)])
                pltpu.sync_copy(cflat, out_c.at[pl.ds(r0, EH)])

        return _gather

    gathers = [_make_gather(k * E) for k in range(NSPLIT)]
    return _interleave, gathers


BM = 1024


def _tower_body(x_ref, c_ref, lab_ref, gb_ref, w1_ref, b1_ref, w2_ref,
                b2_ref, w3_ref, pred_ref, part_ref, acc_ref):
    i = pl.program_id(0)
    x = x_ref[...]
    h = jnp.dot(x, w1_ref[...], preferred_element_type=jnp.float32)
    h = jnp.maximum(h + b1_ref[...], 0.0)
    h = jnp.dot(h, w2_ref[...], preferred_element_type=jnp.float32)
    h = jnp.maximum(h + b2_ref[...], 0.0)
    nn = jnp.dot(h, w3_ref[...], preferred_element_type=jnp.float32)
    logits = jnp.sum(x, axis=1) + nn[:, 0] + gb_ref[0, 0]
    pred_ref[...] = jax.nn.sigmoid(logits)
    craw = jax.nn.sigmoid(jnp.sum(c_ref[...], axis=1)) + 0.5
    xent = (jnp.maximum(logits, 0.0) - logits * lab_ref[...]
            + jnp.log1p(jnp.exp(-jnp.abs(logits))))
    s0 = jnp.sum(xent * craw)
    s1 = jnp.sum(craw)

    @pl.when(i == 0)
    def _():
        acc_ref[0] = 0.0
        acc_ref[1] = 0.0

    acc_ref[0] += s0
    acc_ref[1] += s1

    @pl.when(i == pl.num_programs(0) - 1)
    def _():
        part_ref[0] = acc_ref[0]
        part_ref[1] = acc_ref[1]


def _make_tower(k):
    return pl.pallas_call(
        _tower_body,
        grid=(BCH // BM,),
        in_specs=[
            pl.BlockSpec((BM, S), lambda i: (i, 0)),
            pl.BlockSpec((BM, S), lambda i: (i, 0)),
            pl.BlockSpec((BM,), lambda i, k=k: (k * (BCH // BM) + i,)),
            pl.BlockSpec(memory_space=pltpu.SMEM),
            pl.BlockSpec((S, D1), lambda i: (0, 0)),
            pl.BlockSpec((1, D1), lambda i: (0, 0)),
            pl.BlockSpec((D1, D2), lambda i: (0, 0)),
            pl.BlockSpec((1, D2), lambda i: (0, 0)),
            pl.BlockSpec((D2, 128), lambda i: (0, 0)),
        ],
        out_specs=[
            pl.BlockSpec((BM,), lambda i: (i,)),
            pl.BlockSpec(memory_space=pltpu.SMEM),
        ],
        out_shape=[
            jax.ShapeDtypeStruct((BCH,), jnp.float32),
            jax.ShapeDtypeStruct((2,), jnp.float32),
        ],
        scratch_shapes=[pltpu.SMEM((2,), jnp.float32)],
    )


def kernel(slot_bias_fid_index, label, sparse_bias, certain_bias,
           global_bias, W1, b1, W2, b2, W3, b3):
    idx_flat = slot_bias_fid_index.reshape(-1)
    gb = (global_bias[0] + b3[0]).reshape(1, 1)
    b1r = b1.reshape(1, D1)
    b2r = b2.reshape(1, D2)
    w3c = jnp.zeros((D2, 128), jnp.float32).at[:, 0].set(W3[:, 0])

    interleave, gathers = _get_sc_kernels()
    t2 = interleave(sparse_bias, certain_bias)
    preds = []
    parts = []
    for k in range(NSPLIT):
        out_s, out_c = gathers[k](idx_flat, t2)
        pred_k, part_k = _make_tower(k)(out_s.reshape(BCH, S),
                                        out_c.reshape(BCH, S),
                                        label, gb, W1, b1r, W2, b2r, w3c)
        preds.append(pred_k)
        parts.append(part_k)
    pred = jnp.concatenate(preds)
    ps = jnp.stack(parts)
    s0 = jnp.sum(ps[:, 0])
    s1 = jnp.sum(ps[:, 1])
    loss = B * s0 / s1
    return pred, loss


# speedup vs baseline: 1.1142x; 1.1142x over previous
import functools

import jax
import jax.numpy as jnp
from jax import lax
from jax.experimental import pallas as pl
from jax.experimental.pallas import tpu as pltpu
from jax.experimental.pallas import tpu_sc as plsc

B = 16384
S = 100
D1 = 512
D2 = 256
FID = 1000000

NSPLIT = 4
BCH = B // NSPLIT

NC = 2
NS = 16
NW = NC * NS
E = BCH * S
EW = E // NW
EH = EW // 2

IW = 25
CS = FID // IW
CH2 = CS // 4

_SC_PARAMS = pltpu.CompilerParams(use_tc_tiling_on_sc=False,
                                  needs_layout_passes=False)


@functools.cache
def _get_sc_kernels():
    mesh = plsc.VectorSubcoreMesh(core_axis_name="c", subcore_axis_name="s")

    @functools.partial(
        pl.kernel,
        mesh=mesh,
        out_type=jax.ShapeDtypeStruct((FID, 2), jnp.float32),
        scratch_types=[
            pltpu.VMEM((2 * CH2,), jnp.float32),
            pltpu.VMEM((CH2, 2), jnp.float32),
        ],
        compiler_params=_SC_PARAMS,
    )
    def _interleave(sparse_hbm, certain_hbm, t2, buf2, ibuf):
        wid = lax.axis_index("s") * NC + lax.axis_index("c")

        @pl.when(wid < IW)
        def _():
            lanes = lax.iota(jnp.int32, 16)
            zeros = lanes * 0
            ones = zeros + 1
            for hh in range(4):
                q0 = wid * CS + hh * CH2
                pltpu.sync_copy(sparse_hbm.at[pl.ds(q0, CH2)],
                                buf2.at[pl.ds(0, CH2)])
                pltpu.sync_copy(certain_hbm.at[pl.ds(q0, CH2)],
                                buf2.at[pl.ds(CH2, CH2)])

                def body(j, carry):
                    for u in range(25):
                        base = (j * 25 + u) * 16
                        rows = base + lanes
                        sv = buf2[pl.ds(base, 16)]
                        cv = buf2[pl.ds(CH2 + base, 16)]
                        plsc.store_scatter(ibuf, [rows, zeros], sv)
                        plsc.store_scatter(ibuf, [rows, ones], cv)
                    return carry

                lax.fori_loop(0, CH2 // 400, body, 0)
                pltpu.sync_copy(ibuf, t2.at[pl.ds(q0, CH2)])

    def _make_gather(base):
        @functools.partial(
            pl.kernel,
            mesh=mesh,
            out_type=(jax.ShapeDtypeStruct((E,), jnp.float32),
                      jax.ShapeDtypeStruct((E,), jnp.float32)),
            scratch_types=[
                pltpu.VMEM((EH,), jnp.int32),
                pltpu.VMEM((EH,), jnp.int32),
                pltpu.VMEM((EH, 2), jnp.float32),
                pltpu.VMEM((EH, 2), jnp.float32),
                pltpu.VMEM((EH,), jnp.float32),
                pltpu.VMEM((EH,), jnp.float32),
                pltpu.SemaphoreType.DMA,
                pltpu.SemaphoreType.DMA,
            ],
            compiler_params=_SC_PARAMS,
        )
        def _gather(idx_hbm, t2_hbm, out_s, out_c,
                    idx_v0, idx_v1, obuf0, obuf1, sflat, cflat,
                    sem0, sem1):
            wid = lax.axis_index("s") * NC + lax.axis_index("c")
            lanes = lax.iota(jnp.int32, 16)
            zeros = lanes * 0
            ones = zeros + 1
            idx_vs = (idx_v0, idx_v1)
            obufs = (obuf0, obuf1)
            sems = (sem0, sem1)

            def deint(obuf):
                def body(j, carry):
                    for u in range(8):
                        bb = (j * 8 + u) * 16
                        rows = bb + lanes
                        sv = plsc.load_gather(obuf, [rows, zeros])
                        cv = plsc.load_gather(obuf, [rows, ones])
                        sflat[pl.ds(bb, 16)] = sv
                        cflat[pl.ds(bb, 16)] = cv
                    return carry

                lax.fori_loop(0, EH // 128, body, 0)

            r0w = wid * EW
            pltpu.sync_copy(idx_hbm.at[pl.ds(base + r0w, EH)], idx_v0)
            handles = [pltpu.async_copy(t2_hbm.at[idx_v0], obuf0, sem0)]
            for hh in range(2):
                if hh + 1 < 2:
                    nxt = hh + 1
                    pltpu.sync_copy(
                        idx_hbm.at[pl.ds(base + r0w + nxt * EH, EH)],
                        idx_vs[nxt])
                    handles.append(pltpu.async_copy(
                        t2_hbm.at[idx_vs[nxt]], obufs[nxt], sems[nxt]))
                handles[hh].wait()
                deint(obufs[hh])
                r0 = r0w + hh * EH
                pltpu.sync_copy(sflat, out_s.at[pl.ds(r0, EH)])
                pltpu.sync_copy(cflat, out_c.at[pl.ds(r0, EH)])

        return _gather

    gathers = [_make_gather(k * E) for k in range(NSPLIT)]
    return _interleave, gathers


BM = 1024


def _tower_body(x_ref, c_ref, lab_ref, gb_ref, w1_ref, b1_ref, w2_ref,
                b2_ref, w3_ref, pred_ref, part_ref, acc_ref):
    i = pl.program_id(0)
    x = x_ref[...]
    h = jnp.dot(x, w1_ref[...], preferred_element_type=jnp.float32)
    h = jnp.maximum(h + b1_ref[...], 0.0)
    h = jnp.dot(h, w2_ref[...], preferred_element_type=jnp.float32)
    h = jnp.maximum(h + b2_ref[...], 0.0)
    nn = jnp.dot(h, w3_ref[...], preferred_element_type=jnp.float32)
    logits = jnp.sum(x, axis=1) + nn[:, 0] + gb_ref[0, 0]
    pred_ref[...] = jax.nn.sigmoid(logits)
    craw = jax.nn.sigmoid(jnp.sum(c_ref[...], axis=1)) + 0.5
    xent = (jnp.maximum(logits, 0.0) - logits * lab_ref[...]
            + jnp.log1p(jnp.exp(-jnp.abs(logits))))
    s0 = jnp.sum(xent * craw)
    s1 = jnp.sum(craw)

    @pl.when(i == 0)
    def _():
        acc_ref[0] = 0.0
        acc_ref[1] = 0.0

    acc_ref[0] += s0
    acc_ref[1] += s1

    @pl.when(i == pl.num_programs(0) - 1)
    def _():
        part_ref[0] = acc_ref[0]
        part_ref[1] = acc_ref[1]


def _make_tower(k):
    return pl.pallas_call(
        _tower_body,
        grid=(BCH // BM,),
        in_specs=[
            pl.BlockSpec((BM, S), lambda i: (i, 0)),
            pl.BlockSpec((BM, S), lambda i: (i, 0)),
            pl.BlockSpec((BM,), lambda i, k=k: (k * (BCH // BM) + i,)),
            pl.BlockSpec(memory_space=pltpu.SMEM),
            pl.BlockSpec((S, D1), lambda i: (0, 0)),
            pl.BlockSpec((1, D1), lambda i: (0, 0)),
            pl.BlockSpec((D1, D2), lambda i: (0, 0)),
            pl.BlockSpec((1, D2), lambda i: (0, 0)),
            pl.BlockSpec((D2, 128), lambda i: (0, 0)),
        ],
        out_specs=[
            pl.BlockSpec((BM,), lambda i: (i,)),
            pl.BlockSpec(memory_space=pltpu.SMEM),
        ],
        out_shape=[
            jax.ShapeDtypeStruct((BCH,), jnp.float32),
            jax.ShapeDtypeStruct((2,), jnp.float32),
        ],
        scratch_shapes=[pltpu.SMEM((2,), jnp.float32)],
    )


def kernel(slot_bias_fid_index, label, sparse_bias, certain_bias,
           global_bias, W1, b1, W2, b2, W3, b3):
    idx_flat = slot_bias_fid_index.reshape(-1)
    gb = (global_bias[0] + b3[0]).reshape(1, 1)
    b1r = b1.reshape(1, D1)
    b2r = b2.reshape(1, D2)
    w3c = jnp.zeros((D2, 128), jnp.float32).at[:, 0].set(W3[:, 0])

    interleave, gathers = _get_sc_kernels()
    t2 = interleave(sparse_bias, certain_bias)
    preds = []
    parts = []
    for k in range(NSPLIT):
        idx_k = idx_flat
        if k >= 2:
            idx_k, _ = lax.optimization_barrier((idx_flat, parts[k - 2]))
        out_s, out_c = gathers[k](idx_k, t2)
        pred_k, part_k = _make_tower(k)(out_s.reshape(BCH, S),
                                        out_c.reshape(BCH, S),
                                        label, gb, W1, b1r, W2, b2r, w3c)
        preds.append(pred_k)
        parts.append(part_k)
    pred = jnp.concatenate(preds)
    ps = jnp.stack(parts)
    s0 = jnp.sum(ps[:, 0])
    s1 = jnp.sum(ps[:, 1])
    loss = B * s0 / s1
    return pred, loss
